# Initial kernel scaffold; baseline (speedup 1.0000x reference)
import jax
import jax.numpy as jnp
from jax.experimental import pallas as pl


def kernel(x, z, lengthscale_param):
    raise NotImplementedError("write your pallas kernel here")


# speedup vs baseline: 2.4393x; 2.4393x over previous
import functools

import jax
import jax.numpy as jnp
from jax.experimental import pallas as pl
from jax.experimental.pallas import tpu as pltpu

_GRID_RANGE = ((0.0, 1.0), (0.0, 1.0))
_GRID_SHAPE = (64, 64)


def _setconv_kernel(xt_ref, x_ref, z_ref, ls_ref, ax0_ref, colrep_ref, out_ref):
    p = ls_ref[0, :]
    ls = 1e-5 + jnp.logaddexp(p, 0.0)
    inv = 1.0 / (ls * ls)
    inv0 = inv[0]
    inv1 = inv[1]

    x0_row = xt_ref[0, 0:1, :]
    x1_col = x_ref[0, :, 1:2]
    ax0_col = ax0_ref[...]
    col_rep = colrep_ref[...]

    d0 = ax0_col - x0_row
    a = jnp.exp(-0.5 * inv0 * d0 * d0)

    d1 = x1_col - col_rep
    b = jnp.exp(-0.5 * inv1 * d1 * d1)

    z = z_ref[0]
    z8 = jnp.concatenate([z] * 8, axis=1)
    z_tile = jnp.concatenate([z8] * 8, axis=1)

    t = b * z_tile
    out_ref[0] = jnp.dot(a, t, preferred_element_type=jnp.float32,
                         precision=jax.lax.Precision.HIGHEST)


@functools.partial(jax.jit, static_argnames=())
def kernel(x, z, lengthscale_param):
    m, n, dx = x.shape
    dz = z.shape[-1]
    gi, gj = _GRID_SHAPE

    axes = [jnp.linspace(lo, hi, num, dtype=jnp.float32)
            for (lo, hi), num in zip(_GRID_RANGE, _GRID_SHAPE)]
    grid_pts = jnp.stack(jnp.meshgrid(*axes, indexing='ij'), axis=-1)
    x_grid = jnp.broadcast_to(grid_pts[None], (m, gi, gj, dx))

    xt = jnp.transpose(x, (0, 2, 1))
    ls2 = lengthscale_param.reshape(1, dx)
    ax0 = axes[0].reshape(gi, 1)
    col_rep = jnp.repeat(axes[1], dz).reshape(1, gj * dz)

    out = pl.pallas_call(
        _setconv_kernel,
        grid=(m,),
        in_specs=[
            pl.BlockSpec((1, dx, n), lambda b: (b, 0, 0)),
            pl.BlockSpec((1, n, dx), lambda b: (b, 0, 0)),
            pl.BlockSpec((1, n, dz), lambda b: (b, 0, 0)),
            pl.BlockSpec((1, dx), lambda b: (0, 0)),
            pl.BlockSpec((gi, 1), lambda b: (0, 0)),
            pl.BlockSpec((1, gj * dz), lambda b: (0, 0)),
        ],
        out_specs=pl.BlockSpec((1, gi, gj * dz), lambda b: (b, 0, 0)),
        out_shape=jax.ShapeDtypeStruct((m, gi, gj * dz), jnp.float32),
        compiler_params=pltpu.CompilerParams(
            dimension_semantics=("parallel",),
        ),
    )(xt, x, z, ls2, ax0, col_rep)

    z_grid = out.reshape(m, gi, gj, dz)
    return (x_grid, z_grid)


# speedup vs baseline: 3.7690x; 1.5451x over previous
import functools

import jax
import jax.numpy as jnp
from jax.experimental import pallas as pl
from jax.experimental.pallas import tpu as pltpu

_GRID_RANGE = ((0.0, 1.0), (0.0, 1.0))
_GRID_SHAPE = (64, 64)


def _setconv_kernel(xt_ref, x_ref, z_ref, ls_ref, ax0_ref, colrep_ref, out_ref):
    p = ls_ref[0, :]
    ls = 1e-5 + jnp.logaddexp(p, 0.0)
    inv = 1.0 / (ls * ls)
    inv0 = inv[0]
    inv1 = inv[1]

    x0_row = xt_ref[0, 0:1, :]
    x1_col = x_ref[0, :, 1:2]
    ax0_col = ax0_ref[...]
    col_rep = colrep_ref[...]

    d0 = ax0_col - x0_row
    a = jnp.exp(-0.5 * inv0 * d0 * d0)

    d1 = x1_col - col_rep
    b = jnp.exp(-0.5 * inv1 * d1 * d1)

    z = z_ref[0]
    z8 = jnp.concatenate([z] * 8, axis=1)
    z_tile = jnp.concatenate([z8] * 8, axis=1)

    t = b * z_tile
    out_ref[0] = jnp.dot(a, t, preferred_element_type=jnp.float32,
                         precision=jax.lax.Precision.DEFAULT)


@functools.partial(jax.jit, static_argnames=())
def kernel(x, z, lengthscale_param):
    m, n, dx = x.shape
    dz = z.shape[-1]
    gi, gj = _GRID_SHAPE

    axes = [jnp.linspace(lo, hi, num, dtype=jnp.float32)
            for (lo, hi), num in zip(_GRID_RANGE, _GRID_SHAPE)]
    grid_pts = jnp.stack(jnp.meshgrid(*axes, indexing='ij'), axis=-1)
    x_grid = jnp.broadcast_to(grid_pts[None], (m, gi, gj, dx))

    xt = jnp.transpose(x, (0, 2, 1))
    ls2 = lengthscale_param.reshape(1, dx)
    ax0 = axes[0].reshape(gi, 1)
    col_rep = jnp.repeat(axes[1], dz).reshape(1, gj * dz)

    out = pl.pallas_call(
        _setconv_kernel,
        grid=(m,),
        in_specs=[
            pl.BlockSpec((1, dx, n), lambda b: (b, 0, 0)),
            pl.BlockSpec((1, n, dx), lambda b: (b, 0, 0)),
            pl.BlockSpec((1, n, dz), lambda b: (b, 0, 0)),
            pl.BlockSpec((1, dx), lambda b: (0, 0)),
            pl.BlockSpec((gi, 1), lambda b: (0, 0)),
            pl.BlockSpec((1, gj * dz), lambda b: (0, 0)),
        ],
        out_specs=pl.BlockSpec((1, gi, gj * dz), lambda b: (b, 0, 0)),
        out_shape=jax.ShapeDtypeStruct((m, gi, gj * dz), jnp.float32),
        compiler_params=pltpu.CompilerParams(
            dimension_semantics=("parallel",),
        ),
    )(xt, x, z, ls2, ax0, col_rep)

    z_grid = out.reshape(m, gi, gj, dz)
    return (x_grid, z_grid)
